# baseline (speedup 1.0000x reference)
import functools

import jax
import jax.numpy as jnp
from jax import lax
from jax.experimental import pallas as pl
from jax.experimental.pallas import tpu as pltpu
from jax.experimental.pallas import tpu_sc as plsc

NN = 50000
D = 256
DH = D // 2
K = 6

NC, NS = 2, 16
NW = NC * NS
B = 64
CPW = 1600
NB = CPW // B
NPAD = NW * CPW

_MESH = plsc.VectorSubcoreMesh(
    core_axis_name="c", subcore_axis_name="s", num_cores=NC, num_subcores=NS
)


@functools.partial(
    pl.kernel,
    out_type=jax.ShapeDtypeStruct((NPAD, D), jnp.float32),
    mesh=_MESH,
    scratch_types=[
        pltpu.VMEM((K * CPW,), jnp.int32),
        pltpu.VMEM((2, K, B, DH), jnp.int32),
        pltpu.VMEM((B, D), jnp.float32),
        pltpu.SemaphoreType.DMA,
        pltpu.SemaphoreType.DMA,
    ],
)
def _sc_neighbor_sum(tab_hbm, idxt_hbm, out_hbm, idx_v, bufs, acc, gsem0, gsem1):
    wid = lax.axis_index("s") * NC + lax.axis_index("c")
    base = wid * CPW
    pltpu.sync_copy(idxt_hbm.at[wid], idx_v)
    gsems = (gsem0, gsem1)

    def fire(b, s):
        off = b * B
        for k in range(K):
            pltpu.make_async_copy(
                tab_hbm.at[idx_v.at[pl.ds(k * CPW + off, B)]],
                bufs.at[s, k],
                gsems[s],
            ).start()

    def wait_set(s):
        for k in range(K):
            pltpu.make_async_copy(
                tab_hbm.at[idx_v.at[pl.ds(0, B)]],
                bufs.at[s, k],
                gsems[s],
            ).wait()

    def reduce_set(s):
        def row_body(r, carry):
            for c in range(DH // 16):
                dsl = pl.ds(c * 16, 16)
                v = bufs[s, 0, r, dsl]
                a = lax.bitcast_convert_type(v << 16, jnp.float32)
                b = lax.bitcast_convert_type(v, jnp.float32)
                for k in range(1, K):
                    v = bufs[s, k, r, dsl]
                    a = a + lax.bitcast_convert_type(v << 16, jnp.float32)
                    b = b + lax.bitcast_convert_type(v, jnp.float32)
                acc[r, pl.ds(c * 16, 16)] = a
                acc[r, pl.ds(DH + c * 16, 16)] = b
            return carry
        lax.fori_loop(0, B, row_body, 0)

    def flush(b):
        pltpu.sync_copy(acc, out_hbm.at[pl.ds(base + b * B, B)])

    fire(0, 0)

    def body2(i, carry):
        b0 = i * 2
        for s in (0, 1):
            b = b0 + s

            @pl.when(b + 1 < NB)
            def _():
                fire(b + 1, 1 - s)

            wait_set(s)
            reduce_set(s)
            flush(b)
        return carry

    lax.fori_loop(0, NB // 2, body2, 0)
    wait_set(0)
    reduce_set(0)
    flush(NB - 1)


RB = 400


def _pack_rows(y):
    u = lax.bitcast_convert_type(y, jnp.uint32)

    def rne(t):
        return (t + 0x7FFF + ((t >> 16) & 1)) >> 16

    p = (rne(u[:, DH:]) << 16) | rne(u[:, :DH])
    return lax.bitcast_convert_type(p, jnp.int32)


def _unpack_rows(p):
    return jnp.concatenate(
        [lax.bitcast_convert_type(p << 16, jnp.float32),
         lax.bitcast_convert_type(p & (-65536), jnp.float32)], axis=1)


def _tc_self_body(pt_ref, ws_ref, b_ref, ha_ref):
    x = _unpack_rows(pt_ref[...])
    ha_ref[...] = x + b_ref[...] + jnp.dot(
        x.astype(jnp.bfloat16), ws_ref[...],
        preferred_element_type=jnp.float32)


def _tc_self(pt, ws, b):
    return pl.pallas_call(
        _tc_self_body,
        grid=(NN // RB,),
        in_specs=[
            pl.BlockSpec((RB, DH), lambda i: (i, 0)),
            pl.BlockSpec((D, D), lambda i: (0, 0)),
            pl.BlockSpec((1, D), lambda i: (0, 0)),
        ],
        out_specs=pl.BlockSpec((RB, D), lambda i: (i, 0)),
        out_shape=jax.ShapeDtypeStruct((NN, D), jnp.float32),
    )(pt, ws, b)


def _layer_math(ha_ref, nm_ref, wn_ref, sc_ref, of_ref):
    h = ha_ref[...] + jnp.dot(nm_ref[...].astype(jnp.bfloat16), wn_ref[...],
                              preferred_element_type=jnp.float32)
    mu = jnp.mean(h, axis=-1, keepdims=True)
    xc = h - mu
    var = jnp.mean(xc * xc, axis=-1, keepdims=True)
    y = xc * lax.rsqrt(var + 1e-5) * sc_ref[...] + of_ref[...]
    return jnp.maximum(y, 0.0)


def _tc_mid_body(ha_ref, nm_ref, wn_ref, sc_ref, of_ref, p_ref):
    p_ref[...] = _pack_rows(_layer_math(ha_ref, nm_ref, wn_ref, sc_ref, of_ref))


def _tc_last_body(ha_ref, nm_ref, wn_ref, sc_ref, of_ref, o_ref):
    o_ref[...] = _layer_math(ha_ref, nm_ref, wn_ref, sc_ref, of_ref)


def _tc_layer(body, out_spec, out_shape, ha, nm, wn, scale, offset):
    return pl.pallas_call(
        body,
        grid=(NN // RB,),
        in_specs=[
            pl.BlockSpec((RB, D), lambda i: (i, 0)),
            pl.BlockSpec((RB, D), lambda i: (i, 0)),
            pl.BlockSpec((D, D), lambda i: (0, 0)),
            pl.BlockSpec((1, D), lambda i: (0, 0)),
            pl.BlockSpec((1, D), lambda i: (0, 0)),
        ],
        out_specs=out_spec,
        out_shape=out_shape,
    )(ha, nm, wn, scale, offset)


def kernel(nodes, neighbor_indices, params):
    idx_pad = jnp.zeros((NPAD, K), jnp.int32).at[:NN].set(neighbor_indices)
    idxt = idx_pad.reshape(NW, CPW, K).transpose(0, 2, 1).reshape(NW, K * CPW)
    ptab = _pack_rows(nodes)
    out = None
    for li, p in enumerate(params):
        nmp = _sc_neighbor_sum(ptab, idxt)
        ha = _tc_self(ptab, p["Ws"].astype(jnp.bfloat16),
                      (p["bs"] + p["bn"]).reshape(1, D))
        args = (ha, nmp,
                (p["Wn"] * (1.0 / K)).astype(jnp.bfloat16),
                p["scale"].reshape(1, D), p["offset"].reshape(1, D))
        if li < len(params) - 1:
            ptab = _tc_layer(_tc_mid_body,
                             pl.BlockSpec((RB, DH), lambda i: (i, 0)),
                             jax.ShapeDtypeStruct((NN, DH), jnp.int32), *args)
        else:
            out = _tc_layer(_tc_last_body,
                            pl.BlockSpec((RB, D), lambda i: (i, 0)),
                            jax.ShapeDtypeStruct((NN, D), jnp.float32), *args)
    return out


# speedup vs baseline: 1.1075x; 1.1075x over previous
import functools

import jax
import jax.numpy as jnp
from jax import lax
from jax.experimental import pallas as pl
from jax.experimental.pallas import tpu as pltpu
from jax.experimental.pallas import tpu_sc as plsc

NN = 50000
D = 256
DH = D // 2
K = 6

NC, NS = 2, 16
NW = NC * NS
B = 64
CPW = 1600
NB = CPW // B
NPAD = NW * CPW

_MESH = plsc.VectorSubcoreMesh(
    core_axis_name="c", subcore_axis_name="s", num_cores=NC, num_subcores=NS
)


@functools.partial(
    pl.kernel,
    out_type=jax.ShapeDtypeStruct((NPAD, D), jnp.float32),
    mesh=_MESH,
    scratch_types=[
        pltpu.VMEM((K * CPW,), jnp.int32),
        pltpu.VMEM((2, K, B, DH), jnp.int32),
        pltpu.VMEM((B, D), jnp.float32),
        pltpu.SemaphoreType.DMA,
        pltpu.SemaphoreType.DMA,
    ],
)
def _sc_neighbor_sum(tab_hbm, idxt_hbm, out_hbm, idx_v, bufs, acc, gsem0, gsem1):
    wid = lax.axis_index("s") * NC + lax.axis_index("c")
    base = wid * CPW
    pltpu.sync_copy(idxt_hbm.at[wid], idx_v)
    gsems = (gsem0, gsem1)

    def fire(b, s):
        off = b * B
        for k in range(K):
            pltpu.make_async_copy(
                tab_hbm.at[idx_v.at[pl.ds(k * CPW + off, B)]],
                bufs.at[s, k],
                gsems[s],
            ).start()

    def wait_set(s):
        for k in range(K):
            pltpu.make_async_copy(
                tab_hbm.at[idx_v.at[pl.ds(0, B)]],
                bufs.at[s, k],
                gsems[s],
            ).wait()

    def reduce_set(s):
        def row_body(r, carry):
            for c in range(DH // 16):
                dsl = pl.ds(c * 16, 16)
                v = bufs[s, 0, r, dsl]
                a = lax.bitcast_convert_type(v << 16, jnp.float32)
                b = lax.bitcast_convert_type(v, jnp.float32)
                for k in range(1, K):
                    v = bufs[s, k, r, dsl]
                    a = a + lax.bitcast_convert_type(v << 16, jnp.float32)
                    b = b + lax.bitcast_convert_type(v, jnp.float32)
                acc[r, pl.ds(c * 16, 16)] = a
                acc[r, pl.ds(DH + c * 16, 16)] = b
            return carry
        lax.fori_loop(0, B, row_body, 0)

    def flush(b):
        pltpu.sync_copy(acc, out_hbm.at[pl.ds(base + b * B, B)])

    fire(0, 0)

    def body2(i, carry):
        b0 = i * 2
        for s in (0, 1):
            b = b0 + s

            @pl.when(b + 1 < NB)
            def _():
                fire(b + 1, 1 - s)

            wait_set(s)
            reduce_set(s)
            flush(b)
        return carry

    lax.fori_loop(0, NB // 2, body2, 0)
    wait_set(0)
    reduce_set(0)
    flush(NB - 1)


RB = 400


def _pack_rows(y):
    u = lax.bitcast_convert_type(y, jnp.uint32)

    def rne(t):
        return (t + 0x7FFF + ((t >> 16) & 1)) >> 16

    p = (rne(u[:, DH:]) << 16) | rne(u[:, :DH])
    return lax.bitcast_convert_type(p, jnp.int32)


def _unpack_rows(p):
    return jnp.concatenate(
        [lax.bitcast_convert_type(p << 16, jnp.float32),
         lax.bitcast_convert_type(p & (-65536), jnp.float32)], axis=1)


def _layer_math(pt_ref, nm_ref, ws_ref, wn_ref, b_ref, sc_ref, of_ref):
    x = _unpack_rows(pt_ref[...])
    nm = nm_ref[...]
    h = x + b_ref[...]
    h = h + jnp.dot(x.astype(jnp.bfloat16), ws_ref[...],
                    preferred_element_type=jnp.float32)
    h = h + jnp.dot(nm.astype(jnp.bfloat16), wn_ref[...],
                    preferred_element_type=jnp.float32)
    mu = jnp.mean(h, axis=-1, keepdims=True)
    xc = h - mu
    var = jnp.mean(xc * xc, axis=-1, keepdims=True)
    y = xc * lax.rsqrt(var + 1e-5) * sc_ref[...] + of_ref[...]
    return jnp.maximum(y, 0.0)


def _tc_mid_body(pt_ref, nm_ref, ws_ref, wn_ref, b_ref, sc_ref, of_ref, p_ref):
    p_ref[...] = _pack_rows(_layer_math(pt_ref, nm_ref, ws_ref, wn_ref,
                                        b_ref, sc_ref, of_ref))


def _tc_last_body(pt_ref, nm_ref, ws_ref, wn_ref, b_ref, sc_ref, of_ref, o_ref):
    o_ref[...] = _layer_math(pt_ref, nm_ref, ws_ref, wn_ref,
                             b_ref, sc_ref, of_ref)


def _tc_layer(body, out_spec, out_shape, pt, nm, ws, wn, b, scale, offset):
    return pl.pallas_call(
        body,
        grid=(NN // RB,),
        in_specs=[
            pl.BlockSpec((RB, DH), lambda i: (i, 0)),
            pl.BlockSpec((RB, D), lambda i: (i, 0)),
            pl.BlockSpec((D, D), lambda i: (0, 0)),
            pl.BlockSpec((D, D), lambda i: (0, 0)),
            pl.BlockSpec((1, D), lambda i: (0, 0)),
            pl.BlockSpec((1, D), lambda i: (0, 0)),
            pl.BlockSpec((1, D), lambda i: (0, 0)),
        ],
        out_specs=out_spec,
        out_shape=out_shape,
    )(pt, nm, ws, wn, b, scale, offset)


def kernel(nodes, neighbor_indices, params):
    idx_pad = jnp.zeros((NPAD, K), jnp.int32).at[:NN].set(neighbor_indices)
    idxt = idx_pad.reshape(NW, CPW, K).transpose(0, 2, 1).reshape(NW, K * CPW)
    ptab = _pack_rows(nodes)
    out = None
    for li, p in enumerate(params):
        nmp = _sc_neighbor_sum(ptab, idxt)
        args = (ptab, nmp,
                p["Ws"].astype(jnp.bfloat16),
                (p["Wn"] * (1.0 / K)).astype(jnp.bfloat16),
                (p["bs"] + p["bn"]).reshape(1, D),
                p["scale"].reshape(1, D), p["offset"].reshape(1, D))
        if li < len(params) - 1:
            ptab = _tc_layer(_tc_mid_body,
                             pl.BlockSpec((RB, DH), lambda i: (i, 0)),
                             jax.ShapeDtypeStruct((NN, DH), jnp.int32), *args)
        else:
            out = _tc_layer(_tc_last_body,
                            pl.BlockSpec((RB, D), lambda i: (i, 0)),
                            jax.ShapeDtypeStruct((NN, D), jnp.float32), *args)
    return out


# speedup vs baseline: 1.1195x; 1.0109x over previous
import functools

import jax
import jax.numpy as jnp
from jax import lax
from jax.experimental import pallas as pl
from jax.experimental.pallas import tpu as pltpu
from jax.experimental.pallas import tpu_sc as plsc

NN = 50000
D = 256
DH = D // 2
K = 6

NC, NS = 2, 16
NW = NC * NS
B = 64
CPW = 1600
NB = CPW // B
NPAD = NW * CPW

_MESH = plsc.VectorSubcoreMesh(
    core_axis_name="c", subcore_axis_name="s", num_cores=NC, num_subcores=NS
)


@functools.partial(
    pl.kernel,
    out_type=jax.ShapeDtypeStruct((NPAD, D), jnp.float32),
    mesh=_MESH,
    scratch_types=[
        pltpu.VMEM((K * CPW,), jnp.int32),
        pltpu.VMEM((2, K // 2, 2 * B, DH), jnp.int32),
        pltpu.VMEM((B, D), jnp.float32),
        pltpu.SemaphoreType.DMA,
        pltpu.SemaphoreType.DMA,
    ],
)
def _sc_neighbor_sum(tab_hbm, idxt_hbm, out_hbm, idx_v, bufs, acc, gsem0, gsem1):
    wid = lax.axis_index("s") * NC + lax.axis_index("c")
    base = wid * CPW
    pltpu.sync_copy(idxt_hbm.at[wid], idx_v)
    gsems = (gsem0, gsem1)

    def fire(b, s):
        off = b * 2 * B
        for k2 in range(K // 2):
            pltpu.make_async_copy(
                tab_hbm.at[idx_v.at[pl.ds(k2 * 2 * CPW + off, 2 * B)]],
                bufs.at[s, k2],
                gsems[s],
            ).start()

    def wait_set(s):
        for k2 in range(K // 2):
            pltpu.make_async_copy(
                tab_hbm.at[idx_v.at[pl.ds(0, 2 * B)]],
                bufs.at[s, k2],
                gsems[s],
            ).wait()

    def reduce_set(s):
        def row_body(r, carry):
            for c in range(DH // 16):
                dsl = pl.ds(c * 16, 16)
                v = bufs[s, 0, 2 * r, dsl]
                a = lax.bitcast_convert_type(v << 16, jnp.float32)
                b = lax.bitcast_convert_type(v, jnp.float32)
                for k2 in range(K // 2):
                    for h in range(2):
                        if k2 == 0 and h == 0:
                            continue
                        v = bufs[s, k2, 2 * r + h, dsl]
                        a = a + lax.bitcast_convert_type(v << 16, jnp.float32)
                        b = b + lax.bitcast_convert_type(v, jnp.float32)
                acc[r, pl.ds(c * 16, 16)] = a
                acc[r, pl.ds(DH + c * 16, 16)] = b
            return carry
        lax.fori_loop(0, B, row_body, 0)

    def flush(b):
        pltpu.sync_copy(acc, out_hbm.at[pl.ds(base + b * B, B)])

    fire(0, 0)

    def body2(i, carry):
        b0 = i * 2
        for s in (0, 1):
            b = b0 + s

            @pl.when(b + 1 < NB)
            def _():
                fire(b + 1, 1 - s)

            wait_set(s)
            reduce_set(s)
            flush(b)
        return carry

    lax.fori_loop(0, NB // 2, body2, 0)
    wait_set(0)
    reduce_set(0)
    flush(NB - 1)


RB = 400


def _pack_rows(y):
    u = lax.bitcast_convert_type(y, jnp.uint32)

    def rne(t):
        return (t + 0x7FFF + ((t >> 16) & 1)) >> 16

    p = (rne(u[:, DH:]) << 16) | rne(u[:, :DH])
    return lax.bitcast_convert_type(p, jnp.int32)


def _unpack_rows(p):
    return jnp.concatenate(
        [lax.bitcast_convert_type(p << 16, jnp.float32),
         lax.bitcast_convert_type(p & (-65536), jnp.float32)], axis=1)


def _layer_math(pt_ref, nm_ref, ws_ref, wn_ref, b_ref, sc_ref, of_ref):
    x = _unpack_rows(pt_ref[...])
    nm = nm_ref[...]
    h = x + b_ref[...]
    h = h + jnp.dot(x.astype(jnp.bfloat16), ws_ref[...],
                    preferred_element_type=jnp.float32)
    h = h + jnp.dot(nm.astype(jnp.bfloat16), wn_ref[...],
                    preferred_element_type=jnp.float32)
    mu = jnp.mean(h, axis=-1, keepdims=True)
    xc = h - mu
    var = jnp.mean(xc * xc, axis=-1, keepdims=True)
    y = xc * lax.rsqrt(var + 1e-5) * sc_ref[...] + of_ref[...]
    return jnp.maximum(y, 0.0)


def _tc_mid_body(pt_ref, nm_ref, ws_ref, wn_ref, b_ref, sc_ref, of_ref, p_ref):
    p_ref[...] = _pack_rows(_layer_math(pt_ref, nm_ref, ws_ref, wn_ref,
                                        b_ref, sc_ref, of_ref))


def _tc_last_body(pt_ref, nm_ref, ws_ref, wn_ref, b_ref, sc_ref, of_ref, o_ref):
    o_ref[...] = _layer_math(pt_ref, nm_ref, ws_ref, wn_ref,
                             b_ref, sc_ref, of_ref)


def _tc_layer(body, out_spec, out_shape, pt, nm, ws, wn, b, scale, offset):
    return pl.pallas_call(
        body,
        grid=(NN // RB,),
        in_specs=[
            pl.BlockSpec((RB, DH), lambda i: (i, 0)),
            pl.BlockSpec((RB, D), lambda i: (i, 0)),
            pl.BlockSpec((D, D), lambda i: (0, 0)),
            pl.BlockSpec((D, D), lambda i: (0, 0)),
            pl.BlockSpec((1, D), lambda i: (0, 0)),
            pl.BlockSpec((1, D), lambda i: (0, 0)),
            pl.BlockSpec((1, D), lambda i: (0, 0)),
        ],
        out_specs=out_spec,
        out_shape=out_shape,
    )(pt, nm, ws, wn, b, scale, offset)


def kernel(nodes, neighbor_indices, params):
    idx_pad = jnp.zeros((NPAD, K), jnp.int32).at[:NN].set(neighbor_indices)
    idxt = (idx_pad.reshape(NW, CPW, K // 2, 2).transpose(0, 2, 1, 3)
            .reshape(NW, K * CPW))
    ptab = _pack_rows(nodes)
    out = None
    for li, p in enumerate(params):
        nmp = _sc_neighbor_sum(ptab, idxt)
        args = (ptab, nmp,
                p["Ws"].astype(jnp.bfloat16),
                (p["Wn"] * (1.0 / K)).astype(jnp.bfloat16),
                (p["bs"] + p["bn"]).reshape(1, D),
                p["scale"].reshape(1, D), p["offset"].reshape(1, D))
        if li < len(params) - 1:
            ptab = _tc_layer(_tc_mid_body,
                             pl.BlockSpec((RB, DH), lambda i: (i, 0)),
                             jax.ShapeDtypeStruct((NN, DH), jnp.int32), *args)
        else:
            out = _tc_layer(_tc_last_body,
                            pl.BlockSpec((RB, D), lambda i: (i, 0)),
                            jax.ShapeDtypeStruct((NN, D), jnp.float32), *args)
    return out
